# baseline (speedup 1.0000x reference)
import functools

import jax
import jax.numpy as jnp
import numpy as np
from jax.experimental import pallas as pl

NBLOCKS = 6
NPOINTS = 64
NNEIGHBOR = 16
N_NODES = 10000
N_EDGES = 320000
HIDDEN = 128
NUM_HEADS = 8
N_LAYERS = 4


def _linear(p, x):
    y = x @ p['W'].T
    if 'b' in p:
        y = y + p['b']
    return y


def _layer_norm(p, x):
    mu = jnp.mean(x, axis=-1, keepdims=True)
    var = jnp.var(x, axis=-1, keepdims=True)
    return p['g'] * (x - mu) / jnp.sqrt(var + 1e-5) + p['b']


def _index_points(points, idx):
    return jax.vmap(lambda p, i: p[i])(points, idx)


def _square_distance(src, dst):
    return (jnp.sum(src ** 2, -1)[..., :, None]
            + jnp.sum(dst ** 2, -1)[..., None, :]
            - 2.0 * jnp.einsum('bnc,bmc->bnm', src, dst))


def _fps(xyz, npoint):
    B, N, _ = xyz.shape
    idx_list = []
    distance = jnp.full((B, N), 1e10, dtype=xyz.dtype)
    farthest = jnp.zeros((B,), dtype=jnp.int32)
    for _ in range(npoint):
        idx_list.append(farthest)
        centroid = jnp.take_along_axis(xyz, farthest[:, None, None], axis=1)
        dist = jnp.sum((xyz - centroid) ** 2, -1)
        distance = jnp.minimum(distance, dist)
        farthest = jnp.argmax(distance, axis=-1).astype(jnp.int32)
    return jnp.stack(idx_list, axis=1)


def _sample_group_knn(npoint, nsample, xyz, points):
    fps_idx = _fps(xyz, npoint)
    new_xyz = _index_points(xyz, fps_idx)
    dists = _square_distance(new_xyz, xyz)
    idx = jnp.argsort(dists, axis=-1)[:, :, :nsample]
    grouped_xyz = _index_points(xyz, idx)
    grouped_xyz_norm = grouped_xyz - new_xyz[:, :, None, :]
    grouped_points = _index_points(points, idx)
    new_points = jnp.concatenate([grouped_xyz_norm, grouped_points], axis=-1)
    return new_xyz, new_points


def _set_abstraction(p, npoint, nsample, xyz, points):
    new_xyz, new_points = _sample_group_knn(npoint, nsample, xyz, points)
    hcur = new_points
    for lp in p['mlp']:
        hcur = jax.nn.relu(_linear(lp, hcur))
    return new_xyz, jnp.max(hcur, axis=2)


def _transformer_block(p, k, xyz, features):
    dists = _square_distance(xyz, xyz)
    knn_idx = jnp.argsort(dists, axis=-1)[:, :, :k]
    knn_xyz = _index_points(xyz, knn_idx)
    pre = features
    x = _linear(p['fc1'], features)
    q = _linear(p['w_qs'], x)
    kk = _index_points(_linear(p['w_ks'], x), knn_idx)
    v = _index_points(_linear(p['w_vs'], x), knn_idx)
    pos = xyz[:, :, None, :] - knn_xyz
    pos_enc = _linear(p['fc_delta2'], jax.nn.relu(_linear(p['fc_delta1'], pos)))
    attn = _linear(p['fc_gamma2'], jax.nn.relu(_linear(p['fc_gamma1'], q[:, :, None, :] - kk + pos_enc)))
    attn = jax.nn.softmax(attn / np.sqrt(kk.shape[-1]), axis=-2)
    res = jnp.einsum('bmnf,bmnf->bmf', attn, v + pos_enc)
    res = _linear(p['fc2'], res) + pre
    return res


def _backbone(params, x):
    xyz = x[..., :3]
    feat = _linear(params['fc1_2'], jax.nn.relu(_linear(params['fc1_1'], x)))
    points = _transformer_block(params['transformer1'], NNEIGHBOR, xyz, feat)
    for i in range(NBLOCKS):
        npoint = NPOINTS // 2 ** (i + 1)
        xyz, points = _set_abstraction(params['td'][i], npoint, NNEIGHBOR, xyz, points)
        points = _transformer_block(params['tr'][i], NNEIGHBOR, xyz, points)
    return points


def _graph_layer(p, edge_index, h, num_heads, dh):
    N = h.shape[0]
    Q = _linear(p['Q'], h).reshape(N, num_heads, dh)
    K = _linear(p['K'], h).reshape(N, num_heads, dh)
    V = _linear(p['V'], h).reshape(N, num_heads, dh)
    src = edge_index[0]
    dst = edge_index[1]
    score = jnp.sum(K[src] * Q[dst], axis=-1, keepdims=True)
    score = jnp.exp(jnp.clip(score / np.sqrt(dh), -5.0, 5.0))
    wV = jax.ops.segment_sum(score * V[src], dst, num_segments=N)
    z = jax.ops.segment_sum(score, dst, num_segments=N)
    head_out = wV / z
    h2 = _linear(p['O'], head_out.reshape(N, num_heads * dh))
    h2 = _layer_norm(p['ln1'], h2)
    hf = _linear(p['ffn2'], jax.nn.relu(_linear(p['ffn1'], h2)))
    hf = _layer_norm(p['ln2'], hf)
    return hf


def _identity_kernel(x_ref, o_ref):
    o_ref[...] = x_ref[...]


def _pallas_identity(x):
    return pl.pallas_call(
        _identity_kernel,
        out_shape=jax.ShapeDtypeStruct(x.shape, x.dtype),
    )(x)


def kernel(x, h, e, edge_index, params):
    points = _backbone(params, x)
    points = jnp.squeeze(points, axis=1)
    hh = _linear(params['embedding_h'], h)
    for lp in params['glayers']:
        hh = _graph_layer(lp, edge_index, hh, NUM_HEADS, HIDDEN // NUM_HEADS)
    hg = jnp.mean(hh, axis=0, keepdims=True)
    out = jnp.concatenate([_linear(params['wm1'], points), _linear(params['wm2'], hg)], axis=1)
    return _pallas_identity(out)


# speedup vs baseline: 26.5139x; 26.5139x over previous
import functools

import jax
import jax.numpy as jnp
import numpy as np
from jax import lax
from jax.experimental import pallas as pl
from jax.experimental.pallas import tpu as pltpu
from jax.experimental.pallas import tpu_sc as plsc

NBLOCKS = 6
NPOINTS = 64
NNEIGHBOR = 16
N_NODES = 10000
N_EDGES = 320000
HIDDEN = 128
NUM_HEADS = 8
N_LAYERS = 4
DH = HIDDEN // NUM_HEADS

NC = 2
NS = 16
NW = NC * NS
LANES = 16

EDGE_BLK = 64
N_BLKS = 160
EPW = N_BLKS * EDGE_BLK
E_PAD = EPW * NW
PAIRS = N_BLKS // 2
DST_PAD = 1 << 20
HALF_N = 5120
N_PASS = 2
NZROW = HALF_N // 8
ACC_ROWS = HALF_N + NZROW
APW = ACC_ROWS // NS
APW_CHUNK = 120
SBLK = 2 * EDGE_BLK


_GATHER_DNUMS = lax.GatherDimensionNumbers(
    offset_dims=(), collapsed_slice_dims=(0,), start_index_map=(0,))

_PERM_SRC = np.array([(j % NUM_HEADS) * DH + j // NUM_HEADS
                      for j in range(HIDDEN)], dtype=np.int32)


def _splat_lane(vec, idx):
    return lax.gather(vec, idx[:, None], _GATHER_DNUMS, slice_sizes=(1,),
                      mode=lax.GatherScatterMode.PROMISE_IN_BOUNDS)


def _edge_kernel(k_hbm, q_hbm, v_hbm, src_hbm, dst_hbm, wv_hbm,
                 src0, dst0, srcm0, qm0, dstwz0,
                 src1, dst1, srcm1, qm1, dstwz1,
                 kv0, qv0, vvz0, kv1, qv1, vvz1,
                 stage, acc,
                 semi0, semi1, semg0, semg1, sems0, sems1):
    cid = lax.axis_index("c")
    sid = lax.axis_index("s")
    wid = sid * NC + cid

    lane = lax.iota(jnp.int32, LANES)
    zero16 = jnp.zeros((LANES,), jnp.float32)
    neg1 = jnp.full((LANES,), -1, jnp.int32)
    low = lax.bitwise_and(lane, 7)
    idx_fold = low + 8
    idx_low = low

    slots = (
        (src0, dst0, srcm0, qm0, dstwz0, kv0, qv0, vvz0, semi0, semg0, sems0),
        (src1, dst1, srcm1, qm1, dstwz1, kv1, qv1, vvz1, semi1, semg1, sems1),
    )

    def fire_idx(slot, blk):
        base = wid * EPW + blk * EDGE_BLK
        pltpu.async_copy(src_hbm.at[pl.ds(base, EDGE_BLK)], slot[0], slot[8])
        pltpu.async_copy(dst_hbm.at[pl.ds(base, EDGE_BLK)], slot[1], slot[8])

    def wait_idx(slot):
        pltpu.make_async_copy(
            src_hbm.at[pl.ds(0, EDGE_BLK)], slot[0], slot[8]).wait()
        pltpu.make_async_copy(
            dst_hbm.at[pl.ds(0, EDGE_BLK)], slot[1], slot[8]).wait()

    def gather_specs(slot):
        return (
            (k_hbm.at[plsc.Indices(slot[2], ignored_value=-1)], slot[5], slot[9]),
            (q_hbm.at[plsc.Indices(slot[3], ignored_value=-1)], slot[6], slot[9]),
            (v_hbm.at[plsc.Indices(slot[2], ignored_value=-1)],
             slot[7].at[pl.ds(0, EDGE_BLK), :], slot[9]),
        )

    def scatter_spec(slot):
        return (slot[7], acc.at[plsc.Indices(slot[4], ignored_value=-1)], slot[10])

    def _half_body(half, _h):
        lo = jnp.full((LANES,), half * HALF_N, jnp.int32)

        def _zero_stage(r, _):
            for cpart in range(HIDDEN // LANES):
                stage[r, pl.ds(cpart * LANES, LANES)] = zero16
            return _
        lax.fori_loop(0, APW_CHUNK, _zero_stage, None)
        for j in range(APW // APW_CHUNK):
            pltpu.sync_copy(stage, acc.at[pl.ds(sid * APW + j * APW_CHUNK, APW_CHUNK), :])
        plsc.subcore_barrier()

        fire_idx(slots[0], 0)
        fire_idx(slots[1], 1)

        def _pair_body(pair, _p):
            for si in range(2):
                slot = slots[si]
                blk = 2 * pair + si
                wait_idx(slot)

                @pl.when(pair > 0)
                def _wait_prev_scatter():
                    s, d, sem = scatter_spec(slot)
                    pltpu.make_async_copy(s, d, sem).wait()

                srcm, qm, dstwz = slot[2], slot[3], slot[4]
                for g in range(EDGE_BLK // LANES):
                    sl = pl.ds(g * LANES, LANES)
                    dpart = slot[1][sl]
                    spart = slot[0][sl]
                    dloc = dpart - lo
                    mine = jnp.logical_and(dloc >= 0, dloc < HALF_N)
                    srcm[sl] = jnp.where(mine, spart, neg1)
                    qm[sl] = jnp.where(mine, dpart, neg1)
                    dstwz[sl] = jnp.where(mine, dloc, neg1)
                    dstwz[pl.ds(EDGE_BLK + g * LANES, LANES)] = jnp.where(
                        mine, HALF_N + lax.shift_right_logical(dloc, 3), neg1)

                for s, d, sem in gather_specs(slot):
                    pltpu.async_copy(s, d, sem)

                @pl.when(blk + 2 < N_BLKS)
                def _prefetch_idx():
                    fire_idx(slot, blk + 2)

                for s, d, sem in gather_specs(slot):
                    pltpu.make_async_copy(s, d, sem).wait()

                kv, qv, vvz = slot[5], slot[6], slot[7]

                def _edge_body(ee, _2):
                    sl0 = pl.ds(0, LANES)
                    part = kv[ee, sl0] * qv[ee, sl0]
                    for g in range(1, HIDDEN // LANES):
                        sl = pl.ds(g * LANES, LANES)
                        part = part + kv[ee, sl] * qv[ee, sl]
                    s = (part + _splat_lane(part, idx_fold)) * jnp.float32(0.25)
                    s = jnp.minimum(jnp.maximum(s, jnp.float32(-5.0)),
                                    jnp.float32(5.0))
                    es = jnp.exp(s)
                    m = _splat_lane(es, idx_low)
                    for g in range(HIDDEN // LANES):
                        sl = pl.ds(g * LANES, LANES)
                        vvz[ee, sl] = vvz[ee, sl] * m
                    ze = ee + EDGE_BLK
                    for g in range(HIDDEN // LANES):
                        vvz[ze, pl.ds(g * LANES, LANES)] = zero16
                    rows = _splat_lane(
                        dstwz[pl.ds(lax.mul(lax.div(ee, 16), 16), LANES)],
                        jnp.full((LANES,), lax.rem(ee, 16), jnp.int32))
                    cols = lax.shift_left(lax.bitwise_and(rows, 7), 4) + lane
                    plsc.store_scatter(
                        vvz, [jnp.full((LANES,), ze, jnp.int32), cols], es)
                    return _2
                lax.fori_loop(0, EDGE_BLK, _edge_body, None)

                s, d, sem = scatter_spec(slot)
                pltpu.async_copy(s, d, sem, add=True)
            return _p
        lax.fori_loop(0, PAIRS, _pair_body, None)

        for slot in slots:
            s, d, sem = scatter_spec(slot)
            pltpu.make_async_copy(s, d, sem).wait()
        plsc.subcore_barrier()

        for j in range(APW // APW_CHUNK):
            row0 = sid * APW + j * APW_CHUNK
            pltpu.sync_copy(acc.at[pl.ds(row0, APW_CHUNK), :], stage)
            pltpu.sync_copy(stage, wv_hbm.at[cid, half, pl.ds(row0, APW_CHUNK), :])
        plsc.subcore_barrier()
        return _h
    lax.fori_loop(0, N_PASS, _half_body, None)


_edge_phase = functools.partial(
    pl.kernel,
    out_type=jax.ShapeDtypeStruct((NC, N_PASS, ACC_ROWS, HIDDEN), jnp.float32),
    mesh=plsc.VectorSubcoreMesh(core_axis_name="c", subcore_axis_name="s"),
    compiler_params=pltpu.CompilerParams(needs_layout_passes=False),
    scratch_types=(
        [pltpu.VMEM((EDGE_BLK,), jnp.int32)] * 4
        + [pltpu.VMEM((SBLK,), jnp.int32)]
        + [pltpu.VMEM((EDGE_BLK,), jnp.int32)] * 4
        + [pltpu.VMEM((SBLK,), jnp.int32)]
        + [pltpu.VMEM((EDGE_BLK, HIDDEN), jnp.float32),
           pltpu.VMEM((EDGE_BLK, HIDDEN), jnp.float32),
           pltpu.VMEM((SBLK, HIDDEN), jnp.float32)] * 2
        + [pltpu.VMEM((APW_CHUNK, HIDDEN), jnp.float32),
           pltpu.VMEM_SHARED((ACC_ROWS, HIDDEN), jnp.float32)]
        + [pltpu.SemaphoreType.DMA] * 6
    ),
)(_edge_kernel)


def _linear(p, x):
    y = x @ p['W'].T
    if 'b' in p:
        y = y + p['b']
    return y


def _layer_norm(p, x):
    mu = jnp.mean(x, axis=-1, keepdims=True)
    var = jnp.var(x, axis=-1, keepdims=True)
    return p['g'] * (x - mu) / jnp.sqrt(var + 1e-5) + p['b']


def _index_points(points, idx):
    return jax.vmap(lambda p, i: p[i])(points, idx)


def _square_distance(src, dst):
    return (jnp.sum(src ** 2, -1)[..., :, None]
            + jnp.sum(dst ** 2, -1)[..., None, :]
            - 2.0 * jnp.einsum('bnc,bmc->bnm', src, dst))


def _fps(xyz, npoint):
    B, N, _ = xyz.shape
    idx_list = []
    distance = jnp.full((B, N), 1e10, dtype=xyz.dtype)
    farthest = jnp.zeros((B,), dtype=jnp.int32)
    for _ in range(npoint):
        idx_list.append(farthest)
        centroid = jnp.take_along_axis(xyz, farthest[:, None, None], axis=1)
        dist = jnp.sum((xyz - centroid) ** 2, -1)
        distance = jnp.minimum(distance, dist)
        farthest = jnp.argmax(distance, axis=-1).astype(jnp.int32)
    return jnp.stack(idx_list, axis=1)


def _sample_group_knn(npoint, nsample, xyz, points):
    fps_idx = _fps(xyz, npoint)
    new_xyz = _index_points(xyz, fps_idx)
    dists = _square_distance(new_xyz, xyz)
    idx = jnp.argsort(dists, axis=-1)[:, :, :nsample]
    grouped_xyz = _index_points(xyz, idx)
    grouped_xyz_norm = grouped_xyz - new_xyz[:, :, None, :]
    grouped_points = _index_points(points, idx)
    new_points = jnp.concatenate([grouped_xyz_norm, grouped_points], axis=-1)
    return new_xyz, new_points


def _set_abstraction(p, npoint, nsample, xyz, points):
    new_xyz, new_points = _sample_group_knn(npoint, nsample, xyz, points)
    hcur = new_points
    for lp in p['mlp']:
        hcur = jax.nn.relu(_linear(lp, hcur))
    return new_xyz, jnp.max(hcur, axis=2)


def _transformer_block(p, k, xyz, features):
    dists = _square_distance(xyz, xyz)
    knn_idx = jnp.argsort(dists, axis=-1)[:, :, :k]
    knn_xyz = _index_points(xyz, knn_idx)
    pre = features
    x = _linear(p['fc1'], features)
    q = _linear(p['w_qs'], x)
    kk = _index_points(_linear(p['w_ks'], x), knn_idx)
    v = _index_points(_linear(p['w_vs'], x), knn_idx)
    pos = xyz[:, :, None, :] - knn_xyz
    pos_enc = _linear(p['fc_delta2'], jax.nn.relu(_linear(p['fc_delta1'], pos)))
    attn = _linear(p['fc_gamma2'], jax.nn.relu(_linear(p['fc_gamma1'], q[:, :, None, :] - kk + pos_enc)))
    attn = jax.nn.softmax(attn / np.sqrt(kk.shape[-1]), axis=-2)
    res = jnp.einsum('bmnf,bmnf->bmf', attn, v + pos_enc)
    res = _linear(p['fc2'], res) + pre
    return res


def _backbone(params, x):
    xyz = x[..., :3]
    feat = _linear(params['fc1_2'], jax.nn.relu(_linear(params['fc1_1'], x)))
    points = _transformer_block(params['transformer1'], NNEIGHBOR, xyz, feat)
    for i in range(NBLOCKS):
        npoint = NPOINTS // 2 ** (i + 1)
        xyz, points = _set_abstraction(params['td'][i], npoint, NNEIGHBOR, xyz, points)
        points = _transformer_block(params['tr'][i], NNEIGHBOR, xyz, points)
    return points


def _graph_layer(p, src_pad, dst_pad, h):
    N = h.shape[0]
    Q = _linear({'W': p['Q']['W'][_PERM_SRC]}, h)
    K = _linear({'W': p['K']['W'][_PERM_SRC]}, h)
    V = _linear({'W': p['V']['W'][_PERM_SRC]}, h)
    parts = _edge_phase(K, Q, V, src_pad, dst_pad)
    a = parts[0] + parts[1]
    wV = a[:, :HALF_N].reshape(N_PASS * HALF_N, HIDDEN)[:N_NODES]
    z = a[:, HALF_N:].reshape(N_PASS * HALF_N, LANES)[:N_NODES, :NUM_HEADS]
    head_out = wV / jnp.tile(z, (1, DH))
    h2 = _linear({'W': p['O']['W'][:, _PERM_SRC], 'b': p['O']['b']}, head_out)
    h2 = _layer_norm(p['ln1'], h2)
    hf = _linear(p['ffn2'], jax.nn.relu(_linear(p['ffn1'], h2)))
    hf = _layer_norm(p['ln2'], hf)
    return hf


def kernel(x, h, e, edge_index, params):
    points = _backbone(params, x)
    points = jnp.squeeze(points, axis=1)

    pad = jnp.zeros((E_PAD - N_EDGES,), jnp.int32)
    src_pad = jnp.concatenate([edge_index[0], pad])
    dst_pad = jnp.concatenate([edge_index[1], pad + DST_PAD])

    hh = _linear(params['embedding_h'], h)
    stacked = jax.tree.map(lambda *xs: jnp.stack(xs), *params['glayers'])

    def _scan_body(carry, lp):
        return _graph_layer(lp, src_pad, dst_pad, carry), None

    hh, _ = lax.scan(_scan_body, hh, stacked)
    hg = jnp.mean(hh, axis=0, keepdims=True)
    return jnp.concatenate([_linear(params['wm1'], points), _linear(params['wm2'], hg)], axis=1)
